# baseline (speedup 1.0000x reference)
import math

import jax
import jax.numpy as jnp
from jax.experimental import pallas as pl
from jax.experimental.pallas import tpu as pltpu

_B = 4096
_D = 64
_K = 8192
_C = 1000
_KR = 256
_NRARE = 4
_TEMP = 1.5
_INV_LOG_C = 1.0 / math.log(float(_C))

_KBLK = 2048
_NKB = _K // _KBLK
_BBLK = 1024
_NBB = _B // _BBLK
_XBLK = _B // _NKB


def _body(p_ref, w_ref, x_ref, r_ref, g_ref, out_ref, a_ref, m4_ref):
    i = pl.program_id(0)

    @pl.when(i < _NKB)
    def _phase1():
        p = p_ref[...]
        pn = p * jax.lax.rsqrt(jnp.sum(p * p, axis=1, keepdims=True) + 1e-12)
        part = jax.lax.dot_general(
            pn, w_ref[...], (((0,), (0,)), ((), ())),
            preferred_element_type=jnp.float32)

        @pl.when(i == 0)
        def _init():
            a_ref[...] = part

        @pl.when(i != 0)
        def _acc():
            a_ref[...] += part

        x = x_ref[pl.ds(i * _XBLK, _XBLK), :]
        fn = x * jax.lax.rsqrt(jnp.sum(x * x, axis=1, keepdims=True) + 1e-12)
        r = r_ref[...]
        rn = r * jax.lax.rsqrt(jnp.sum(r * r, axis=1, keepdims=True) + 1e-12)
        sim = jax.lax.dot_general(
            fn, rn, (((1,), (1,)), ((), ())),
            preferred_element_type=jnp.float32)
        m4_ref[pl.ds(i * _XBLK, _XBLK), :] = jnp.concatenate(
            [jnp.max(sim[:, j * _KR:(j + 1) * _KR], axis=1, keepdims=True)
             for j in range(_NRARE)], axis=1)

    @pl.when(i >= _NKB)
    def _phase2():
        b = i - _NKB
        x = x_ref[pl.ds(b * _BBLK, _BBLK), :]
        fn = x * jax.lax.rsqrt(jnp.sum(x * x, axis=1, keepdims=True) + 1e-12)
        ls = jnp.dot(fn, a_ref[...], preferred_element_type=jnp.float32)

        z = ls * (1.0 / _TEMP)
        ez = jnp.exp(z)
        ones = jnp.ones((_C, 128), jnp.float32)
        se = jax.lax.dot_general(
            ez, ones, (((1,), (0,)), ((), ())),
            preferred_element_type=jnp.float32)[:, 0:1]
        sz = jax.lax.dot_general(
            ez * z, ones, (((1,), (0,)), ((), ())),
            preferred_element_type=jnp.float32)[:, 0:1]
        ent = jnp.log(se) - sz / se
        u = ent * _INV_LOG_C

        m4 = m4_ref[pl.ds(b * _BBLK, _BBLK), :]
        g4 = g_ref[0:1, 0:_NRARE]
        out_ref[...] = ls
        out_ref[:, 0:_NRARE] = ls[:, 0:_NRARE] + u * (m4 * g4)


def kernel(x, shared_prototypes, W_shared_to_class, rare_prototypes, rarity_factor):
    rare_flat = rare_prototypes.reshape(_NRARE * _KR, _D)
    gates = rarity_factor.reshape(1, _C)

    logits = pl.pallas_call(
        _body,
        grid=(_NKB + _NBB,),
        in_specs=[
            pl.BlockSpec((_KBLK, _D), lambda i: (jnp.minimum(i, _NKB - 1), 0)),
            pl.BlockSpec((_KBLK, _C), lambda i: (jnp.minimum(i, _NKB - 1), 0)),
            pl.BlockSpec((_B, _D), lambda i: (0, 0)),
            pl.BlockSpec((_NRARE * _KR, _D), lambda i: (0, 0)),
            pl.BlockSpec((1, _C), lambda i: (0, 0)),
        ],
        out_specs=pl.BlockSpec((_BBLK, _C), lambda i: (jnp.maximum(i - _NKB, 0), 0)),
        out_shape=jax.ShapeDtypeStruct((_B, _C), jnp.float32),
        scratch_shapes=[
            pltpu.VMEM((_D, _C), jnp.float32),
            pltpu.VMEM((_B, _NRARE), jnp.float32),
        ],
    )(shared_prototypes, W_shared_to_class, x, rare_flat, gates)

    return logits


# speedup vs baseline: 1.0160x; 1.0160x over previous
import math

import jax
import jax.numpy as jnp
from jax.experimental import pallas as pl
from jax.experimental.pallas import tpu as pltpu

_B = 4096
_D = 64
_K = 8192
_C = 1000
_KR = 256
_NRARE = 4
_TEMP = 1.5
_INV_LOG_C = 1.0 / math.log(float(_C))

_KBLK = 2048
_NKB = _K // _KBLK
_BBLK = 2048
_NBB = _B // _BBLK
_XBLK = _B // _NKB


def _body(p_ref, w_ref, x_ref, r_ref, g_ref, out_ref, a_ref, m4_ref):
    i = pl.program_id(0)

    @pl.when(i < _NKB)
    def _phase1():
        p = p_ref[...]
        pn = p * jax.lax.rsqrt(jnp.sum(p * p, axis=1, keepdims=True) + 1e-12)
        part = jax.lax.dot_general(
            pn, w_ref[...], (((0,), (0,)), ((), ())),
            preferred_element_type=jnp.float32)

        @pl.when(i == 0)
        def _init():
            a_ref[...] = part

        @pl.when(i != 0)
        def _acc():
            a_ref[...] += part

        x = x_ref[pl.ds(i * _XBLK, _XBLK), :]
        fn = x * jax.lax.rsqrt(jnp.sum(x * x, axis=1, keepdims=True) + 1e-12)
        r = r_ref[...]
        rn = r * jax.lax.rsqrt(jnp.sum(r * r, axis=1, keepdims=True) + 1e-12)
        sim = jax.lax.dot_general(
            fn, rn, (((1,), (1,)), ((), ())),
            preferred_element_type=jnp.float32)
        m4_ref[pl.ds(i * _XBLK, _XBLK), :] = jnp.concatenate(
            [jnp.max(sim[:, j * _KR:(j + 1) * _KR], axis=1, keepdims=True)
             for j in range(_NRARE)], axis=1)

    @pl.when(i >= _NKB)
    def _phase2():
        b = i - _NKB
        x = x_ref[pl.ds(b * _BBLK, _BBLK), :]
        fn = x * jax.lax.rsqrt(jnp.sum(x * x, axis=1, keepdims=True) + 1e-12)
        ls = jnp.dot(fn, a_ref[...], preferred_element_type=jnp.float32)

        z = ls * (1.0 / _TEMP)
        ez = jnp.exp(z)
        se = jnp.sum(ez, axis=1, keepdims=True)
        sz = jnp.sum(ez * z, axis=1, keepdims=True)
        ent = jnp.log(se) - sz / se
        u = ent * _INV_LOG_C

        m4 = m4_ref[pl.ds(b * _BBLK, _BBLK), :]
        g4 = g_ref[0:1, 0:_NRARE]
        out_ref[...] = ls
        out_ref[:, 0:_NRARE] = ls[:, 0:_NRARE] + u * (m4 * g4)


def kernel(x, shared_prototypes, W_shared_to_class, rare_prototypes, rarity_factor):
    rare_flat = rare_prototypes.reshape(_NRARE * _KR, _D)
    gates = rarity_factor.reshape(1, _C)

    logits = pl.pallas_call(
        _body,
        grid=(_NKB + _NBB,),
        in_specs=[
            pl.BlockSpec((_KBLK, _D), lambda i: (jnp.minimum(i, _NKB - 1), 0)),
            pl.BlockSpec((_KBLK, _C), lambda i: (jnp.minimum(i, _NKB - 1), 0)),
            pl.BlockSpec((_B, _D), lambda i: (0, 0)),
            pl.BlockSpec((_NRARE * _KR, _D), lambda i: (0, 0)),
            pl.BlockSpec((1, _C), lambda i: (0, 0)),
        ],
        out_specs=pl.BlockSpec((_BBLK, _C), lambda i: (jnp.maximum(i - _NKB, 0), 0)),
        out_shape=jax.ShapeDtypeStruct((_B, _C), jnp.float32),
        scratch_shapes=[
            pltpu.VMEM((_D, _C), jnp.float32),
            pltpu.VMEM((_B, _NRARE), jnp.float32),
        ],
    )(shared_prototypes, W_shared_to_class, x, rare_flat, gates)

    return logits


# speedup vs baseline: 1.0221x; 1.0060x over previous
import math

import jax
import jax.numpy as jnp
from jax.experimental import pallas as pl
from jax.experimental.pallas import tpu as pltpu

_B = 4096
_D = 64
_K = 8192
_C = 1000
_KR = 256
_NRARE = 4
_TEMP = 1.5
_INV_LOG_C = 1.0 / math.log(float(_C))

_KBLK = 2048
_NKB = _K // _KBLK
_BBLK = 1024
_NBB = _B // _BBLK
_XBLK = _B // _NKB


def _body(p_ref, w_ref, x_ref, r_ref, g_ref, out_ref, a_ref, m4_ref):
    i = pl.program_id(0)

    @pl.when(i < _NKB)
    def _phase1():
        p = p_ref[...]
        pn = p * jax.lax.rsqrt(jnp.sum(p * p, axis=1, keepdims=True) + 1e-12)
        part = jax.lax.dot_general(
            pn, w_ref[...], (((0,), (0,)), ((), ())),
            preferred_element_type=jnp.float32)

        @pl.when(i == 0)
        def _init():
            a_ref[...] = part

        @pl.when(i != 0)
        def _acc():
            a_ref[...] += part

        x = x_ref[pl.ds(i * _XBLK, _XBLK), :]
        fn = x * jax.lax.rsqrt(jnp.sum(x * x, axis=1, keepdims=True) + 1e-12)
        r = r_ref[...]
        rn = r * jax.lax.rsqrt(jnp.sum(r * r, axis=1, keepdims=True) + 1e-12)
        sim = jax.lax.dot_general(
            fn, rn, (((1,), (1,)), ((), ())),
            preferred_element_type=jnp.float32)
        m4_ref[pl.ds(i * _XBLK, _XBLK), :] = jnp.concatenate(
            [jnp.max(sim[:, j * _KR:(j + 1) * _KR], axis=1, keepdims=True)
             for j in range(_NRARE)], axis=1)

    @pl.when(i >= _NKB)
    def _phase2():
        b = i - _NKB
        x = x_ref[pl.ds(b * _BBLK, _BBLK), :]
        fn = x * jax.lax.rsqrt(jnp.sum(x * x, axis=1, keepdims=True) + 1e-12)
        ls = jnp.dot(fn, a_ref[...], preferred_element_type=jnp.float32)

        z = ls * (1.0 / _TEMP)
        ez = jnp.exp(z)
        se = jnp.sum(ez, axis=1, keepdims=True)
        sz = jnp.sum(ez * z, axis=1, keepdims=True)
        ent = jnp.log(se) - sz / se
        u = ent * _INV_LOG_C

        m4 = m4_ref[pl.ds(b * _BBLK, _BBLK), :]
        g4 = g_ref[0:1, 0:_NRARE]
        out_ref[...] = ls
        out_ref[:, 0:_NRARE] = ls[:, 0:_NRARE] + u * (m4 * g4)


def kernel(x, shared_prototypes, W_shared_to_class, rare_prototypes, rarity_factor):
    rare_flat = rare_prototypes.reshape(_NRARE * _KR, _D)
    gates = rarity_factor.reshape(1, _C)

    logits = pl.pallas_call(
        _body,
        grid=(_NKB + _NBB,),
        in_specs=[
            pl.BlockSpec((_KBLK, _D), lambda i: (jnp.minimum(i, _NKB - 1), 0)),
            pl.BlockSpec((_KBLK, _C), lambda i: (jnp.minimum(i, _NKB - 1), 0)),
            pl.BlockSpec((_B, _D), lambda i: (0, 0)),
            pl.BlockSpec((_NRARE * _KR, _D), lambda i: (0, 0)),
            pl.BlockSpec((1, _C), lambda i: (0, 0)),
        ],
        out_specs=pl.BlockSpec((_BBLK, _C), lambda i: (jnp.maximum(i - _NKB, 0), 0)),
        out_shape=jax.ShapeDtypeStruct((_B, _C), jnp.float32),
        scratch_shapes=[
            pltpu.VMEM((_D, _C), jnp.float32),
            pltpu.VMEM((_B, _NRARE), jnp.float32),
        ],
    )(shared_prototypes, W_shared_to_class, x, rare_flat, gates)

    return logits
